# baseline (device time: 38978 ns/iter reference)
import functools

import jax
import jax.numpy as jnp
from jax import lax
from jax.experimental import pallas as pl
from jax.experimental.pallas import tpu as pltpu

N_DEV = 32
N_STAGES = 5
B, SQ, DMODEL = 2, 256, 512
HQ, DH = 4, 64
NQB = SQ // 64


def kernel(x, Wq, K_ext, V_ext, Wo):
    def body(
        x_ref, wq_ref, k_ref, v_ref, wo_ref, out_ref,
        num_ref, den_ref, nsend_ref, nrecv_ref, drecv_ref,
        nsend_sems, nrecv_sems, dsend_sems, drecv_sems,
    ):
        my = lax.axis_index("i")
        partners = [jnp.bitwise_xor(my, 1 << s) for s in range(N_STAGES)]

        def compute_partial(b):
            qp = jnp.dot(
                x_ref[b], wq_ref[...], preferred_element_type=jnp.float32
            )
            for h in range(HQ):
                for qb in range(NQB):
                    qs = slice(qb * 64, (qb + 1) * 64)
                    Qb = qp[qs, h * 64:(h + 1) * 64]
                    Kb = k_ref[b, qs, h, :]
                    Vb = v_ref[b, qs, h, :]
                    sT = lax.dot_general(
                        Kb, Qb, (((1,), (1,)), ((), ())),
                        preferred_element_type=jnp.float32,
                    )
                    wT = jnp.exp(sT * 0.125)
                    numT = lax.dot_general(
                        Vb, wT, (((0,), (0,)), ((), ())),
                        preferred_element_type=jnp.float32,
                    )
                    num_ref[b, h, :, qs] = numT
                    den_ref[b * HQ + h:b * HQ + h + 1, qs] = jnp.sum(
                        wT, axis=0, keepdims=True
                    )

        def finalize(b):
            acc = jnp.zeros((SQ, DMODEL), dtype=jnp.float32)
            for h in range(HQ):
                ctxT = num_ref[b, h] / den_ref[b * HQ + h:b * HQ + h + 1, :]
                acc = acc + lax.dot_general(
                    ctxT, wo_ref[h * 64:(h + 1) * 64, :],
                    (((0,), (0,)), ((), ())),
                    preferred_element_type=jnp.float32,
                )
            out_ref[b] = acc

        def make_n(c, s):
            return pltpu.make_async_remote_copy(
                src_ref=nsend_ref.at[c],
                dst_ref=nrecv_ref.at[s, c],
                send_sem=nsend_sems.at[s, c],
                recv_sem=nrecv_sems.at[s, c],
                device_id=(partners[s],),
                device_id_type=pl.DeviceIdType.MESH,
            )

        def make_d(c, s):
            return pltpu.make_async_remote_copy(
                src_ref=den_ref.at[pl.ds(c * HQ, HQ)],
                dst_ref=drecv_ref.at[s, pl.ds(c * HQ, HQ)],
                send_sem=dsend_sems.at[s, c],
                recv_sem=drecv_sems.at[s, c],
                device_id=(partners[s],),
                device_id_type=pl.DeviceIdType.MESH,
            )

        rdma_n = {}
        rdma_d = {}

        compute_partial(0)

        barrier_sem = pltpu.get_barrier_semaphore()
        for p in partners:
            pl.semaphore_signal(
                barrier_sem, inc=1, device_id=(p,),
                device_id_type=pl.DeviceIdType.MESH,
            )
        pl.semaphore_wait(barrier_sem, N_STAGES)

        nsend_ref[0] = num_ref[0].astype(jnp.bfloat16)
        rdma_n[(0, 0)] = make_n(0, 0)
        rdma_n[(0, 0)].start()
        rdma_d[(0, 0)] = make_d(0, 0)
        rdma_d[(0, 0)].start()

        compute_partial(1)
        nsend_ref[1] = num_ref[1].astype(jnp.bfloat16)
        rdma_n[(1, 0)] = make_n(1, 0)
        rdma_n[(1, 0)].start()
        rdma_d[(1, 0)] = make_d(1, 0)
        rdma_d[(1, 0)].start()

        for s in range(N_STAGES):
            for c in range(B):
                rdma_n[(c, s)].wait()
                rdma_d[(c, s)].wait()
                num_ref[c] = num_ref[c] + nrecv_ref[s, c].astype(jnp.float32)
                dr = slice(c * HQ, (c + 1) * HQ)
                den_ref[dr, :] = den_ref[dr, :] + drecv_ref[s, dr, :]
                if s + 1 < N_STAGES:
                    nsend_ref[c] = num_ref[c].astype(jnp.bfloat16)
                    rdma_n[(c, s + 1)] = make_n(c, s + 1)
                    rdma_n[(c, s + 1)].start()
                    rdma_d[(c, s + 1)] = make_d(c, s + 1)
                    rdma_d[(c, s + 1)].start()
                else:
                    finalize(c)

        @functools.partial(
            pl.run_scoped, exit_sem=pltpu.SemaphoreType.REGULAR
        )
        def _(exit_sem):
            for p in partners:
                pl.semaphore_signal(
                    exit_sem, inc=1, device_id=(p,),
                    device_id_type=pl.DeviceIdType.MESH,
                )
            pl.semaphore_wait(exit_sem, N_STAGES)

    return pl.pallas_call(
        body,
        out_shape=jax.ShapeDtypeStruct((B, SQ, DMODEL), jnp.float32),
        in_specs=[pl.BlockSpec(memory_space=pltpu.VMEM)] * 5,
        out_specs=pl.BlockSpec(memory_space=pltpu.VMEM),
        scratch_shapes=[
            pltpu.VMEM((B, HQ, DH, SQ), jnp.float32),
            pltpu.VMEM((B * HQ, SQ), jnp.float32),
            pltpu.VMEM((B, HQ, DH, SQ), jnp.bfloat16),
            pltpu.VMEM((N_STAGES, B, HQ, DH, SQ), jnp.bfloat16),
            pltpu.VMEM((N_STAGES, B * HQ, SQ), jnp.float32),
            pltpu.SemaphoreType.DMA((N_STAGES, B)),
            pltpu.SemaphoreType.DMA((N_STAGES, B)),
            pltpu.SemaphoreType.DMA((N_STAGES, B)),
            pltpu.SemaphoreType.DMA((N_STAGES, B)),
        ],
        compiler_params=pltpu.CompilerParams(collective_id=0),
    )(x, Wq, K_ext, V_ext, Wo)


# device time: 38792 ns/iter; 1.0048x vs baseline; 1.0048x over previous
import functools

import jax
import jax.numpy as jnp
from jax import lax
from jax.experimental import pallas as pl
from jax.experimental.pallas import tpu as pltpu

N_DEV = 32
N_STAGES = 5
B, SQ, DMODEL = 2, 256, 512
HQ, DH = 4, 64
NQB = SQ // 64
ROWS = HQ * DH

RS_ORDER = (0, 3, 1, 2, 4)
RS_SIZES = tuple(ROWS >> (i + 1) for i in range(N_STAGES))


def kernel(x, Wq, K_ext, V_ext, Wo):
    def body(
        x_ref, wq_ref, k_ref, v_ref, wo_ref, out_ref,
        num_ref, den_ref, nsend_ref, nrecv_ref, drecv_ref,
        nsend_sems, nrecv_sems, dsend_sems, drecv_sems,
    ):
        my = lax.axis_index("i")
        partners = [jnp.bitwise_xor(my, 1 << k) for k in range(N_STAGES)]
        bits = [jnp.bitwise_and(lax.shift_right_logical(my, k), 1)
                for k in range(N_STAGES)]

        def compute_partial(b):
            qp = jnp.dot(
                x_ref[b], wq_ref[...], preferred_element_type=jnp.float32
            )
            for h in range(HQ):
                r0 = (b * HQ + h) * DH
                for qb in range(NQB):
                    qs = slice(qb * 64, (qb + 1) * 64)
                    Qb = qp[qs, h * 64:(h + 1) * 64]
                    Kb = k_ref[b, qs, h, :]
                    Vb = v_ref[b, qs, h, :]
                    sT = lax.dot_general(
                        Kb, Qb, (((1,), (1,)), ((), ())),
                        preferred_element_type=jnp.float32,
                    )
                    wT = jnp.exp(sT * 0.125)
                    numT = lax.dot_general(
                        Vb, wT, (((0,), (0,)), ((), ())),
                        preferred_element_type=jnp.float32,
                    )
                    num_ref[r0:r0 + DH, qs] = numT
                    den_ref[b * HQ + h:b * HQ + h + 1, qs] = jnp.sum(
                        wT, axis=0, keepdims=True
                    )

        def finalize(b):
            acc = jnp.zeros((SQ, DMODEL), dtype=jnp.float32)
            for h in range(HQ):
                r0 = (b * HQ + h) * DH
                ctxT = num_ref[r0:r0 + DH, :] / den_ref[
                    b * HQ + h:b * HQ + h + 1, :
                ]
                acc = acc + lax.dot_general(
                    ctxT, wo_ref[h * 64:(h + 1) * 64, :],
                    (((0,), (0,)), ((), ())),
                    preferred_element_type=jnp.float32,
                )
            out_ref[b] = acc

        def make_n(c, t, k, rows):
            return pltpu.make_async_remote_copy(
                src_ref=nsend_ref.at[c, pl.ds(0, rows)],
                dst_ref=nrecv_ref.at[t, c, pl.ds(0, rows)],
                send_sem=nsend_sems.at[t, c],
                recv_sem=nrecv_sems.at[t, c],
                device_id=(partners[k],),
                device_id_type=pl.DeviceIdType.MESH,
            )

        def make_d(c, s):
            return pltpu.make_async_remote_copy(
                src_ref=den_ref.at[pl.ds(c * HQ, HQ)],
                dst_ref=drecv_ref.at[s, pl.ds(c * HQ, HQ)],
                send_sem=dsend_sems.at[s, c],
                recv_sem=drecv_sems.at[s, c],
                device_id=(partners[RS_ORDER[s]],),
                device_id_type=pl.DeviceIdType.MESH,
            )

        rdma_n = {}
        rdma_d = {}
        off = [jnp.int32(0), jnp.int32(ROWS)]

        def launch(c, t):
            if t < N_STAGES:
                s = t
                half = RS_SIZES[s]
                bit = bits[RS_ORDER[s]]
                send_off = off[c] + (1 - bit) * half
                nsend_ref[c, :half, :] = num_ref[
                    pl.ds(send_off, half), :
                ].astype(jnp.bfloat16)
                rdma_n[(c, t)] = make_n(c, t, RS_ORDER[s], half)
                rdma_n[(c, t)].start()
                rdma_d[(c, s)] = make_d(c, s)
                rdma_d[(c, s)].start()
            else:
                j = t - N_STAGES
                rows = RS_SIZES[N_STAGES - 1 - j]
                nsend_ref[c, :rows, :] = num_ref[
                    pl.ds(off[c], rows), :
                ].astype(jnp.bfloat16)
                rdma_n[(c, t)] = make_n(c, t, RS_ORDER[N_STAGES - 1 - j], rows)
                rdma_n[(c, t)].start()

        def consume(c, t):
            rdma_n[(c, t)].wait()
            if t < N_STAGES:
                s = t
                half = RS_SIZES[s]
                bit = bits[RS_ORDER[s]]
                keep_off = off[c] + bit * half
                num_ref[pl.ds(keep_off, half), :] = num_ref[
                    pl.ds(keep_off, half), :
                ] + nrecv_ref[t, c, :half, :].astype(jnp.float32)
                rdma_d[(c, s)].wait()
                dr = slice(c * HQ, (c + 1) * HQ)
                den_ref[dr, :] = den_ref[dr, :] + drecv_ref[s, dr, :]
                off[c] = keep_off
            else:
                j = t - N_STAGES
                rows = RS_SIZES[N_STAGES - 1 - j]
                bit = bits[RS_ORDER[N_STAGES - 1 - j]]
                base = off[c] - bit * rows
                p_off = base + (1 - bit) * rows
                num_ref[pl.ds(p_off, rows), :] = nrecv_ref[
                    t, c, :rows, :
                ].astype(jnp.float32)
                off[c] = base

        N_STEPS = 2 * N_STAGES

        compute_partial(0)

        barrier_sem = pltpu.get_barrier_semaphore()
        for p in partners:
            pl.semaphore_signal(
                barrier_sem, inc=1, device_id=(p,),
                device_id_type=pl.DeviceIdType.MESH,
            )
        pl.semaphore_wait(barrier_sem, N_STAGES)

        launch(0, 0)
        compute_partial(1)
        launch(1, 0)

        for t in range(N_STEPS):
            for c in range(B):
                consume(c, t)
                if t + 1 < N_STEPS:
                    launch(c, t + 1)
                else:
                    finalize(c)

        @functools.partial(
            pl.run_scoped, exit_sem=pltpu.SemaphoreType.REGULAR
        )
        def _(exit_sem):
            for p in partners:
                pl.semaphore_signal(
                    exit_sem, inc=1, device_id=(p,),
                    device_id_type=pl.DeviceIdType.MESH,
                )
            pl.semaphore_wait(exit_sem, N_STAGES)

    return pl.pallas_call(
        body,
        out_shape=jax.ShapeDtypeStruct((B, SQ, DMODEL), jnp.float32),
        in_specs=[pl.BlockSpec(memory_space=pltpu.VMEM)] * 5,
        out_specs=pl.BlockSpec(memory_space=pltpu.VMEM),
        scratch_shapes=[
            pltpu.VMEM((B * ROWS, SQ), jnp.float32),
            pltpu.VMEM((B * HQ, SQ), jnp.float32),
            pltpu.VMEM((B, ROWS // 2, SQ), jnp.bfloat16),
            pltpu.VMEM((2 * N_STAGES, B, ROWS // 2, SQ), jnp.bfloat16),
            pltpu.VMEM((N_STAGES, B * HQ, SQ), jnp.float32),
            pltpu.SemaphoreType.DMA((2 * N_STAGES, B)),
            pltpu.SemaphoreType.DMA((2 * N_STAGES, B)),
            pltpu.SemaphoreType.DMA((N_STAGES, B)),
            pltpu.SemaphoreType.DMA((N_STAGES, B)),
        ],
        compiler_params=pltpu.CompilerParams(collective_id=0),
    )(x, Wq, K_ext, V_ext, Wo)
